# Initial kernel scaffold; baseline (speedup 1.0000x reference)
import jax
import jax.numpy as jnp
from jax.experimental import pallas as pl


def kernel(x, edge_index, Wl1, bl1, Wr1, gamma, beta, Wl2, bl2, Wr2):
    raise NotImplementedError("write your pallas kernel here")


# speedup vs baseline: 5.5971x; 5.5971x over previous
import functools

import jax
import jax.numpy as jnp
from jax import lax
from jax.experimental import pallas as pl
from jax.experimental.pallas import tpu as pltpu
from jax.experimental.pallas import tpu_sc as plsc

NC = 2
NS = 16
NW = NC * NS
K = 80
L = 16


@functools.lru_cache(maxsize=None)
def _sc_segsum(n, e, d, with_counts):
    epw = e // NW
    nchunk = epw // K
    rquot = 8 * (n // (NS * 8))
    tail = n - NS * rquot
    assert epw % K == 0 and tail % 8 == 0 and tail <= rquot and n % L == 0

    mesh = plsc.VectorSubcoreMesh(core_axis_name="c", subcore_axis_name="s")
    out_type = [jax.ShapeDtypeStruct((NC, n, d), jnp.float32)]
    scratch = {
        "src_v": pltpu.VMEM((K,), jnp.int32),
        "dst_v": pltpu.VMEM((K,), jnp.int32),
        "rows_v": pltpu.VMEM((K, d), jnp.float32),
        "acc_s": pltpu.VMEM_SHARED((n, d), jnp.float32),
        "sem": pltpu.SemaphoreType.DMA,
    }
    if with_counts:
        out_type.append(jax.ShapeDtypeStruct((NW * n,), jnp.float32))
        scratch["cnt_v"] = pltpu.VMEM((n,), jnp.float32)

    def body(x_hbm, src_hbm, dst_hbm, zeros_hbm, sums_hbm, cnts_hbm=None,
             *, src_v, dst_v, rows_v, acc_s, sem, cnt_v=None):
        c = lax.axis_index("c")
        s = lax.axis_index("s")
        w = c * NS + s
        rbase = s * rquot

        def over_rows(fn):
            fn(rbase, rquot)
            if tail:
                @pl.when(s == NS - 1)
                def _():
                    fn(NS * rquot, tail)

        over_rows(lambda b, m: pltpu.sync_copy(
            zeros_hbm.at[pl.ds(0, m)], acc_s.at[pl.ds(b, m)]))
        if with_counts:
            def zero_cnt(i, carry):
                cnt_v[pl.ds(i * L, L)] = jnp.zeros((L,), jnp.float32)
                return carry
            lax.fori_loop(0, n // L, zero_cnt, 0)
        plsc.subcore_barrier()

        ebase = w * epw
        ones16 = jnp.ones((L,), jnp.float32)

        def chunk(i, carry):
            off = ebase + i * K
            pltpu.sync_copy(src_hbm.at[pl.ds(off, K)], src_v)
            pltpu.sync_copy(dst_hbm.at[pl.ds(off, K)], dst_v)
            pltpu.async_copy(x_hbm.at[src_v], rows_v, sem).wait()
            pltpu.sync_copy(rows_v, acc_s.at[dst_v], add=True)
            if with_counts:
                for j in range(K // L):
                    idx = dst_v[pl.ds(j * L, L)]
                    plsc.addupdate_scatter(cnt_v, [idx], ones16)
            return carry

        lax.fori_loop(0, nchunk, chunk, 0)
        plsc.subcore_barrier()
        over_rows(lambda b, m: pltpu.sync_copy(
            acc_s.at[pl.ds(b, m)], sums_hbm.at[c].at[pl.ds(b, m)]))
        if with_counts:
            pltpu.sync_copy(cnt_v, cnts_hbm.at[pl.ds(w * n, n)])

    if with_counts:
        def body_wc(x, src, dst, z, sums, cnts, **scr):
            body(x, src, dst, z, sums, cnts, **scr)
        fn = body_wc
    else:
        def body_nc(x, src, dst, z, sums, **scr):
            body(x, src, dst, z, sums, None, **scr)
        fn = body_nc

    return pl.kernel(
        fn, out_type=out_type, mesh=mesh, scratch_types=scratch,
        compiler_params=pltpu.CompilerParams(needs_layout_passes=False))


def _tc1_body(sums_ref, cnts_ref, x_ref, wl_ref, bl_ref, wr_ref, g_ref,
              b_ref, o_ref, cnt_ref):
    cnt = jnp.maximum(jnp.sum(cnts_ref[...], axis=0), 1.0)[:, None]
    cnt_ref[...] = cnt
    ssum = sums_ref[0] + sums_ref[1]
    mean = ssum / cnt
    h = (jnp.dot(mean, wl_ref[...], preferred_element_type=jnp.float32)
         + bl_ref[...]
         + jnp.dot(x_ref[...], wr_ref[...], preferred_element_type=jnp.float32))
    mu = jnp.mean(h, axis=0, keepdims=True)
    var = jnp.mean((h - mu) ** 2, axis=0, keepdims=True)
    hn = (h - mu) * lax.rsqrt(var + 1e-5) * g_ref[...] + b_ref[...]
    o_ref[...] = jnp.where(hn >= 0, hn, 0.01 * hn)


def _tc2_body(sums_ref, cnt_ref, h_ref, wl_ref, bl_ref, wr_ref, o_ref):
    ssum = sums_ref[0] + sums_ref[1]
    mean = ssum / cnt_ref[...]
    o_ref[...] = (jnp.dot(mean, wl_ref[...], preferred_element_type=jnp.float32)
                  + bl_ref[...]
                  + jnp.dot(h_ref[...], wr_ref[...],
                            preferred_element_type=jnp.float32))


def kernel(x, edge_index, Wl1, bl1, Wr1, gamma, beta, Wl2, bl2, Wr2):
    n, d = x.shape
    e = edge_index.shape[1]
    src = edge_index[0].astype(jnp.int32)
    dst = edge_index[1].astype(jnp.int32)
    rquot = 8 * (n // (NS * 8))
    zeros = jnp.zeros((rquot, d), jnp.float32)

    sums1, cnts = _sc_segsum(n, e, d, True)(x, src, dst, zeros)
    h, cnt_col = pl.pallas_call(
        _tc1_body,
        out_shape=[jax.ShapeDtypeStruct((n, d), jnp.float32),
                   jax.ShapeDtypeStruct((n, 1), jnp.float32)],
    )(sums1, cnts.reshape(NW, n), x, Wl1, bl1.reshape(1, -1), Wr1,
      gamma.reshape(1, -1), beta.reshape(1, -1))
    (sums2,) = _sc_segsum(n, e, d, False)(h, src, dst, zeros)
    out = pl.pallas_call(
        _tc2_body,
        out_shape=jax.ShapeDtypeStruct((n, d), jnp.float32),
    )(sums2, cnt_col, h, Wl2, bl2.reshape(1, -1), Wr2)
    return out


# speedup vs baseline: 10.7164x; 1.9146x over previous
import functools

import jax
import jax.numpy as jnp
from jax import lax
from jax.experimental import pallas as pl
from jax.experimental.pallas import tpu as pltpu
from jax.experimental.pallas import tpu_sc as plsc

NC = 2
NS = 16
NW = NC * NS
K = 80
L = 16


@functools.lru_cache(maxsize=None)
def _sc_segsum(n, e, d, with_counts):
    epw = e // NW
    nchunk = epw // K
    rquot = 8 * (n // (NS * 8))
    tail = n - NS * rquot
    assert epw % K == 0 and tail % 8 == 0 and tail <= rquot and n % L == 0

    mesh = plsc.VectorSubcoreMesh(core_axis_name="c", subcore_axis_name="s")
    out_type = [jax.ShapeDtypeStruct((NC, n, d), jnp.float32)]
    scratch = {
        "src_c": pltpu.VMEM((2, K), jnp.int32),
        "dst_b": pltpu.VMEM((2, K), jnp.int32),
        "rows0": pltpu.VMEM((K, d), jnp.float32),
        "rows1": pltpu.VMEM((K, d), jnp.float32),
        "acc_s": pltpu.VMEM_SHARED((n, d), jnp.float32),
        "gsem0": pltpu.SemaphoreType.DMA,
        "gsem1": pltpu.SemaphoreType.DMA,
        "ssem0": pltpu.SemaphoreType.DMA,
        "ssem1": pltpu.SemaphoreType.DMA,
        "isem0": pltpu.SemaphoreType.DMA,
        "isem1": pltpu.SemaphoreType.DMA,
    }
    if with_counts:
        out_type.append(jax.ShapeDtypeStruct((NW * n,), jnp.float32))
        scratch["cnt_v"] = pltpu.VMEM((n,), jnp.float32)

    def body(x_hbm, src_hbm, dst_hbm, zeros_hbm, sums_hbm, cnts_hbm=None,
             *, src_c, dst_b, rows0, rows1, acc_s, gsem0, gsem1, ssem0,
             ssem1, isem0, isem1, cnt_v=None):
        c = lax.axis_index("c")
        s = lax.axis_index("s")
        w = c * NS + s
        rbase = s * rquot
        ebase = w * epw

        def over_rows(fn):
            fn(rbase, rquot)
            if tail:
                @pl.when(s == NS - 1)
                def _():
                    fn(NS * rquot, tail)

        over_rows(lambda b, m: pltpu.sync_copy(
            zeros_hbm.at[pl.ds(0, m)], acc_s.at[pl.ds(b, m)]))
        if with_counts:
            def zero_cnt(i, carry):
                cnt_v[pl.ds(i * L, L)] = jnp.zeros((L,), jnp.float32)
                return carry
            lax.fori_loop(0, n // L, zero_cnt, 0)
        plsc.subcore_barrier()

        ones16 = jnp.ones((L,), jnp.float32)
        isems = (isem0, isem1)

        def idx_load(i, p):
            off = ebase + i * K
            sem = isems[p]
            pltpu.async_copy(src_hbm.at[pl.ds(off, K)], src_c.at[p], sem)
            pltpu.async_copy(dst_hbm.at[pl.ds(off, K)], dst_b.at[p], sem)

        def iwait(i, p):
            off = ebase + i * K
            sem = isems[p]
            pltpu.make_async_copy(src_hbm.at[pl.ds(off, K)], src_c.at[p],
                                  sem).wait()
            pltpu.make_async_copy(dst_hbm.at[pl.ds(off, K)], dst_b.at[p],
                                  sem).wait()

        def gather(p, buf, sem):
            return pltpu.async_copy(x_hbm.at[src_c.at[p]], buf, sem)

        def gwait(p, buf, sem):
            pltpu.make_async_copy(x_hbm.at[src_c.at[p]], buf, sem).wait()

        def scatter(p, buf, sem):
            return pltpu.async_copy(buf, acc_s.at[dst_b.at[p]], sem,
                                    add=True)

        def swait(p, buf, sem):
            pltpu.make_async_copy(buf, acc_s.at[dst_b.at[p]], sem).wait()

        def counts(p):
            if with_counts:
                for j in range(K // L):
                    idx = dst_b[p, pl.ds(j * L, L)]
                    plsc.addupdate_scatter(cnt_v, [idx], ones16)

        assert nchunk % 2 == 1 and nchunk >= 3
        npair = nchunk // 2
        idx_load(0, 0)
        iwait(0, 0)
        gather(0, rows0, gsem0)

        def pair(t, carry):
            i0 = 2 * t

            @pl.when(t > 0)
            def _():
                swait(1, rows1, ssem1)
            idx_load(i0 + 1, 1)
            counts(0)
            iwait(i0 + 1, 1)
            gather(1, rows1, gsem1)
            gwait(0, rows0, gsem0)
            scatter(0, rows0, ssem0).wait()
            idx_load(i0 + 2, 0)
            counts(1)
            iwait(i0 + 2, 0)
            gwait(1, rows1, gsem1)
            scatter(1, rows1, ssem1)
            gather(0, rows0, gsem0)
            return carry

        lax.fori_loop(0, npair, pair, 0)
        swait(1, rows1, ssem1)
        counts(0)
        gwait(0, rows0, gsem0)
        scatter(0, rows0, ssem0).wait()
        plsc.subcore_barrier()
        over_rows(lambda b, m: pltpu.sync_copy(
            acc_s.at[pl.ds(b, m)], sums_hbm.at[c].at[pl.ds(b, m)]))
        if with_counts:
            pltpu.sync_copy(cnt_v, cnts_hbm.at[pl.ds(w * n, n)])

    if with_counts:
        def body_wc(x, src, dst, z, sums, cnts, **scr):
            body(x, src, dst, z, sums, cnts, **scr)
        fn = body_wc
    else:
        def body_nc(x, src, dst, z, sums, **scr):
            body(x, src, dst, z, sums, None, **scr)
        fn = body_nc

    return pl.kernel(
        fn, out_type=out_type, mesh=mesh, scratch_types=scratch,
        compiler_params=pltpu.CompilerParams(needs_layout_passes=False))


def _tc1_body(sums_ref, cnts_ref, x_ref, wl_ref, bl_ref, wr_ref, g_ref,
              b_ref, o_ref, cnt_ref):
    cnt = jnp.maximum(jnp.sum(cnts_ref[...], axis=0), 1.0)[:, None]
    cnt_ref[...] = cnt
    ssum = sums_ref[0] + sums_ref[1]
    mean = ssum / cnt
    h = (jnp.dot(mean, wl_ref[...], preferred_element_type=jnp.float32)
         + bl_ref[...]
         + jnp.dot(x_ref[...], wr_ref[...], preferred_element_type=jnp.float32))
    mu = jnp.mean(h, axis=0, keepdims=True)
    var = jnp.mean((h - mu) ** 2, axis=0, keepdims=True)
    hn = (h - mu) * lax.rsqrt(var + 1e-5) * g_ref[...] + b_ref[...]
    o_ref[...] = jnp.where(hn >= 0, hn, 0.01 * hn)


def _tc2_body(sums_ref, cnt_ref, h_ref, wl_ref, bl_ref, wr_ref, o_ref):
    ssum = sums_ref[0] + sums_ref[1]
    mean = ssum / cnt_ref[...]
    o_ref[...] = (jnp.dot(mean, wl_ref[...], preferred_element_type=jnp.float32)
                  + bl_ref[...]
                  + jnp.dot(h_ref[...], wr_ref[...],
                            preferred_element_type=jnp.float32))


def kernel(x, edge_index, Wl1, bl1, Wr1, gamma, beta, Wl2, bl2, Wr2):
    n, d = x.shape
    e = edge_index.shape[1]
    src = edge_index[0].astype(jnp.int32)
    dst = edge_index[1].astype(jnp.int32)
    rquot = 8 * (n // (NS * 8))
    zeros = jnp.zeros((rquot, d), jnp.float32)

    sums1, cnts = _sc_segsum(n, e, d, True)(x, src, dst, zeros)
    h, cnt_col = pl.pallas_call(
        _tc1_body,
        out_shape=[jax.ShapeDtypeStruct((n, d), jnp.float32),
                   jax.ShapeDtypeStruct((n, 1), jnp.float32)],
    )(sums1, cnts.reshape(NW, n), x, Wl1, bl1.reshape(1, -1), Wr1,
      gamma.reshape(1, -1), beta.reshape(1, -1))
    (sums2,) = _sc_segsum(n, e, d, False)(h, src, dst, zeros)
    out = pl.pallas_call(
        _tc2_body,
        out_shape=jax.ShapeDtypeStruct((n, d), jnp.float32),
    )(sums2, cnt_col, h, Wl2, bl2.reshape(1, -1), Wr2)
    return out
